# Initial kernel scaffold; baseline (speedup 1.0000x reference)
import jax
import jax.numpy as jnp
from jax.experimental import pallas as pl


def kernel(feat0, feat1, feat2, feat3, lat_w0, lat_b0, fpn_w0, fpn_b0, lat_w1, lat_b1, fpn_w1, fpn_b1, lat_w2, lat_b2, fpn_w2, fpn_b2, lat_w3, lat_b3, fpn_w3, fpn_b3):
    raise NotImplementedError("write your pallas kernel here")


# speedup vs baseline: 1.0400x; 1.0400x over previous
import functools

import jax
import jax.numpy as jnp
from jax.experimental import pallas as pl
from jax.experimental.pallas import tpu as pltpu

_C = 256
_VMEM_LIMIT = 44 * 1024 * 1024


def _rup(x, m):
    return ((x + m - 1) // m) * m


def _big_divisor(n, cap, mult=1):
    best = None
    for d in range(1, n + 1):
        if n % d == 0 and d <= cap and d % mult == 0:
            best = d
    return best if best is not None else n


def _lat_kernel(x_ref, w_ref, b_ref, o_ref):
    xb = x_ref[0].astype(jnp.bfloat16)
    acc = jax.lax.dot_general(
        xb, w_ref[...], (((0,), (0,)), ((), ())),
        preferred_element_type=jnp.float32)
    o_ref[0] = (acc + b_ref[...]).astype(o_ref.dtype)


def _lateral(x, w_bf16, b):
    N, Cin, H, W = x.shape
    hw = H * W
    x3 = x.reshape(N, Cin, hw)
    tq = _big_divisor(hw, max(256, (2 * 1024 * 1024) // (Cin * 4)), mult=256)
    out = pl.pallas_call(
        _lat_kernel,
        out_shape=jax.ShapeDtypeStruct((N, hw, _C), jnp.bfloat16),
        grid=(N, hw // tq),
        in_specs=[
            pl.BlockSpec((1, Cin, tq), lambda n, q: (n, 0, q)),
            pl.BlockSpec((Cin, _C), lambda n, q: (0, 0)),
            pl.BlockSpec((1, _C), lambda n, q: (0, 0)),
        ],
        out_specs=pl.BlockSpec((1, tq, _C), lambda n, q: (n, q, 0)),
        compiler_params=pltpu.CompilerParams(
            dimension_semantics=("parallel", "parallel"),
            vmem_limit_bytes=_VMEM_LIMIT),
    )(x3, w_bf16, b.reshape(1, _C))
    return out.reshape(N, H, W, _C)


def _merge_kernel(big_ref, small_ref, o_ref):
    th = big_ref.shape[1]
    h2, w2, c = small_ref.shape[1], small_ref.shape[2], small_ref.shape[3]
    s = small_ref[0]
    sh = jnp.broadcast_to(s[:, None], (h2, 2, w2, c)).reshape(th, w2, c)
    o_ref[0] = big_ref[0] + pltpu.repeat(sh, 2, axis=2)


def _merge2x(big, small):
    N, H, W, _ = big.shape
    w2 = W // 2
    row_bytes = w2 * 2 * _C * 2
    th = _big_divisor(H, max(2, (2 * 1024 * 1024) // row_bytes), mult=2)
    if H // th < 2 and H >= 4:
        th = H // 2
    bigv = big.reshape(N, H, w2, 2 * _C)
    out = pl.pallas_call(
        _merge_kernel,
        out_shape=jax.ShapeDtypeStruct((N, H, w2, 2 * _C), jnp.bfloat16),
        grid=(N, H // th),
        in_specs=[
            pl.BlockSpec((1, th, w2, 2 * _C), lambda n, i: (n, i, 0, 0)),
            pl.BlockSpec((1, th // 2, w2, _C), lambda n, i: (n, i, 0, 0)),
        ],
        out_specs=pl.BlockSpec((1, th, w2, 2 * _C), lambda n, i: (n, i, 0, 0)),
        compiler_params=pltpu.CompilerParams(
            dimension_semantics=("parallel", "parallel"),
            vmem_limit_bytes=_VMEM_LIMIT),
    )(bigv, small)
    return out.reshape(N, H, W, _C)


def _conv3_kernel(x_hbm, w_ref, b_ref, o_ref, xbuf, sem, *, wp, th, ch):
    n = pl.program_id(0)
    i = pl.program_id(1)
    nt = pl.num_programs(1)
    rows = th * wp
    slab = (th + 4) * wp
    slot = i % 2

    def fetch(tile, s):
        pltpu.make_async_copy(
            x_hbm.at[n, pl.ds(tile * rows, slab), :], xbuf.at[s], sem.at[s]
        ).start()

    @pl.when(i == 0)
    def _():
        fetch(0, 0)

    @pl.when(i + 1 < nt)
    def _():
        fetch(i + 1, 1 - slot)

    pltpu.make_async_copy(
        x_hbm.at[n, pl.ds(i * rows, slab), :], xbuf.at[slot], sem.at[slot]
    ).wait()

    c = o_ref.shape[2]
    for m in range(rows // ch):
        base = m * ch
        taps = jnp.concatenate(
            [xbuf[slot, pl.ds(base + ky * wp + kx, ch), :]
             for ky in range(3) for kx in range(3)], axis=-1)
        acc = jnp.dot(taps, w_ref[...], preferred_element_type=jnp.float32)
        o_ref[0, base:base + ch, :] = acc + b_ref[...]


def _conv3x3(x, w9_bf16, b):
    N, H, W, _ = x.shape
    wp = _rup(W + 2, 8)
    th = min(16, H)
    rows = th * wp
    ch = _big_divisor(rows, 192, mult=8)
    xp = jnp.pad(x, ((0, 0), (1, 3), (1, wp - W - 1), (0, 0)))
    xf = xp.reshape(N, (H + 4) * wp, _C)
    out = pl.pallas_call(
        functools.partial(_conv3_kernel, wp=wp, th=th, ch=ch),
        out_shape=jax.ShapeDtypeStruct((N, H * wp, _C), jnp.float32),
        grid=(N, H // th),
        in_specs=[
            pl.BlockSpec(memory_space=pl.ANY),
            pl.BlockSpec((9 * _C, _C), lambda n, i: (0, 0)),
            pl.BlockSpec((1, _C), lambda n, i: (0, 0)),
        ],
        out_specs=pl.BlockSpec((1, rows, _C), lambda n, i: (n, i, 0)),
        scratch_shapes=[
            pltpu.VMEM((2, (th + 4) * wp, _C), jnp.bfloat16),
            pltpu.SemaphoreType.DMA((2,)),
        ],
        compiler_params=pltpu.CompilerParams(
            dimension_semantics=("parallel", "arbitrary"),
            vmem_limit_bytes=_VMEM_LIMIT),
    )(xf, w9_bf16, b.reshape(1, _C))
    o4 = out.reshape(N, H, wp, _C)
    return jnp.transpose(o4[:, :, :W, :], (0, 3, 1, 2))


@jax.jit
def _fpn(feats, lat_ws, lat_bs, fpn_ws, fpn_bs):
    nlev = len(feats)
    lats = [
        _lateral(feats[i], lat_ws[i].astype(jnp.bfloat16), lat_bs[i])
        for i in range(nlev)
    ]
    for i in range(nlev - 1, 0, -1):
        lats[i - 1] = _merge2x(lats[i - 1], lats[i])
    outs = []
    for i in range(nlev):
        w9 = fpn_ws[i].reshape(9 * _C, _C).astype(jnp.bfloat16)
        outs.append(_conv3x3(lats[i], w9, fpn_bs[i]))
    return outs


def kernel(feat0, feat1, feat2, feat3,
           lat_w0, lat_b0, fpn_w0, fpn_b0,
           lat_w1, lat_b1, fpn_w1, fpn_b1,
           lat_w2, lat_b2, fpn_w2, fpn_b2,
           lat_w3, lat_b3, fpn_w3, fpn_b3):
    return _fpn(
        [feat0, feat1, feat2, feat3],
        [lat_w0, lat_w1, lat_w2, lat_w3],
        [lat_b0, lat_b1, lat_b2, lat_b3],
        [fpn_w0, fpn_w1, fpn_w2, fpn_w3],
        [fpn_b0, fpn_b1, fpn_b2, fpn_b3],
    )


# speedup vs baseline: 1.0717x; 1.0305x over previous
import functools

import jax
import jax.numpy as jnp
from jax.experimental import pallas as pl
from jax.experimental.pallas import tpu as pltpu

_C = 256
_VMEM_LIMIT = 44 * 1024 * 1024


def _rup(x, m):
    return ((x + m - 1) // m) * m


def _big_divisor(n, cap, mult=1):
    best = None
    for d in range(1, n + 1):
        if n % d == 0 and d <= cap and d % mult == 0:
            best = d
    return best if best is not None else n


def _lat_kernel(x_ref, w_ref, b_ref, o_ref):
    xb = x_ref[0].astype(jnp.bfloat16)
    acc = jax.lax.dot_general(
        xb, w_ref[...], (((0,), (0,)), ((), ())),
        preferred_element_type=jnp.float32)
    o_ref[0] = (acc + b_ref[...]).astype(o_ref.dtype)


def _lateral(x, w_bf16, b):
    N, Cin, H, W = x.shape
    hw = H * W
    x3 = x.reshape(N, Cin, hw)
    tq = _big_divisor(hw, max(256, (2 * 1024 * 1024) // (Cin * 4)), mult=256)
    out = pl.pallas_call(
        _lat_kernel,
        out_shape=jax.ShapeDtypeStruct((N, hw, _C), jnp.bfloat16),
        grid=(N, hw // tq),
        in_specs=[
            pl.BlockSpec((1, Cin, tq), lambda n, q: (n, 0, q)),
            pl.BlockSpec((Cin, _C), lambda n, q: (0, 0)),
            pl.BlockSpec((1, _C), lambda n, q: (0, 0)),
        ],
        out_specs=pl.BlockSpec((1, tq, _C), lambda n, q: (n, q, 0)),
        compiler_params=pltpu.CompilerParams(
            dimension_semantics=("parallel", "parallel"),
            vmem_limit_bytes=_VMEM_LIMIT),
    )(x3, w_bf16, b.reshape(1, _C))
    return out.reshape(N, H, W, _C)


def _merge_kernel(big_ref, small_ref, o_ref):
    th = big_ref.shape[1]
    h2, w2, c = small_ref.shape[1], small_ref.shape[2], small_ref.shape[3]
    s = small_ref[0]
    sh = jnp.broadcast_to(s[:, None], (h2, 2, w2, c)).reshape(th, w2, c)
    o_ref[0] = big_ref[0] + pltpu.repeat(sh, 2, axis=2)


def _merge2x(big, small):
    N, H, W, _ = big.shape
    w2 = W // 2
    row_bytes = w2 * 2 * _C * 2
    th = _big_divisor(H, max(2, (2 * 1024 * 1024) // row_bytes), mult=2)
    if H // th < 2 and H >= 4:
        th = H // 2
    bigv = big.reshape(N, H, w2, 2 * _C)
    out = pl.pallas_call(
        _merge_kernel,
        out_shape=jax.ShapeDtypeStruct((N, H, w2, 2 * _C), jnp.bfloat16),
        grid=(N, H // th),
        in_specs=[
            pl.BlockSpec((1, th, w2, 2 * _C), lambda n, i: (n, i, 0, 0)),
            pl.BlockSpec((1, th // 2, w2, _C), lambda n, i: (n, i, 0, 0)),
        ],
        out_specs=pl.BlockSpec((1, th, w2, 2 * _C), lambda n, i: (n, i, 0, 0)),
        compiler_params=pltpu.CompilerParams(
            dimension_semantics=("parallel", "parallel"),
            vmem_limit_bytes=_VMEM_LIMIT),
    )(bigv, small)
    return out.reshape(N, H, W, _C)


def _conv3_kernel(x_hbm, w_ref, b_ref, o_ref, xbuf, sem, *, wp, w, th, cr):
    n = pl.program_id(0)
    i = pl.program_id(1)
    nt = pl.num_programs(1)
    rows = th * wp
    slab = (th + 4) * wp
    slot = i % 2

    def fetch(tile, s):
        pltpu.make_async_copy(
            x_hbm.at[n, pl.ds(tile * rows, slab), :], xbuf.at[s], sem.at[s]
        ).start()

    @pl.when(i == 0)
    def _():
        fetch(0, 0)

    @pl.when(i + 1 < nt)
    def _():
        fetch(i + 1, 1 - slot)

    pltpu.make_async_copy(
        x_hbm.at[n, pl.ds(i * rows, slab), :], xbuf.at[slot], sem.at[slot]
    ).wait()

    ch = cr * wp
    for m in range(th // cr):
        base = m * ch
        taps = jnp.concatenate(
            [xbuf[slot, pl.ds(base + ky * wp + kx, ch), :]
             for ky in range(3) for kx in range(3)], axis=-1)
        acc = jnp.dot(taps, w_ref[...], preferred_element_type=jnp.float32)
        acc_t = jnp.transpose(acc + b_ref[...], (1, 0))
        valid = jnp.concatenate(
            [acc_t[:, r * wp:r * wp + w] for r in range(cr)], axis=-1)
        o_ref[0, :, m * cr * w:(m + 1) * cr * w] = valid


def _conv3x3(x, w9_bf16, b):
    N, H, W, _ = x.shape
    wp = _rup(W + 2, 8)
    th = min(16, H)
    cr = 2
    while cr < th and (2 * cr) * wp <= 400:
        cr *= 2
    xp = jnp.pad(x, ((0, 0), (1, 3), (1, wp - W - 1), (0, 0)))
    xf = xp.reshape(N, (H + 4) * wp, _C)
    out = pl.pallas_call(
        functools.partial(_conv3_kernel, wp=wp, w=W, th=th, cr=cr),
        out_shape=jax.ShapeDtypeStruct((N, _C, H * W), jnp.float32),
        grid=(N, H // th),
        in_specs=[
            pl.BlockSpec(memory_space=pl.ANY),
            pl.BlockSpec((9 * _C, _C), lambda n, i: (0, 0)),
            pl.BlockSpec((1, _C), lambda n, i: (0, 0)),
        ],
        out_specs=pl.BlockSpec((1, _C, th * W), lambda n, i: (n, 0, i)),
        scratch_shapes=[
            pltpu.VMEM((2, (th + 4) * wp, _C), jnp.bfloat16),
            pltpu.SemaphoreType.DMA((2,)),
        ],
        compiler_params=pltpu.CompilerParams(
            dimension_semantics=("parallel", "arbitrary"),
            vmem_limit_bytes=_VMEM_LIMIT),
    )(xf, w9_bf16, b.reshape(1, _C))
    return out.reshape(N, _C, H, W)


@jax.jit
def _fpn(feats, lat_ws, lat_bs, fpn_ws, fpn_bs):
    nlev = len(feats)
    lats = [
        _lateral(feats[i], lat_ws[i].astype(jnp.bfloat16), lat_bs[i])
        for i in range(nlev)
    ]
    for i in range(nlev - 1, 0, -1):
        lats[i - 1] = _merge2x(lats[i - 1], lats[i])
    outs = []
    for i in range(nlev):
        w9 = fpn_ws[i].reshape(9 * _C, _C).astype(jnp.bfloat16)
        outs.append(_conv3x3(lats[i], w9, fpn_bs[i]))
    return outs


def kernel(feat0, feat1, feat2, feat3,
           lat_w0, lat_b0, fpn_w0, fpn_b0,
           lat_w1, lat_b1, fpn_w1, fpn_b1,
           lat_w2, lat_b2, fpn_w2, fpn_b2,
           lat_w3, lat_b3, fpn_w3, fpn_b3):
    return _fpn(
        [feat0, feat1, feat2, feat3],
        [lat_w0, lat_w1, lat_w2, lat_w3],
        [lat_b0, lat_b1, lat_b2, lat_b3],
        [fpn_w0, fpn_w1, fpn_w2, fpn_w3],
        [fpn_b0, fpn_b1, fpn_b2, fpn_b3],
    )
